# Initial kernel scaffold; baseline (speedup 1.0000x reference)
import functools

import jax
import jax.numpy as jnp
from jax import lax
from jax.experimental import pallas as pl
from jax.experimental.pallas import tpu as pltpu
from jax.experimental.pallas import tpu_sc as plsc

B = 16384
V = 8
L = 16
NC, NS = 2, 16
NW = NC * NS
BPW = B // NW
CHUNK = 128
NCH = BPW // CHUNK
GRP = BPW // L


@functools.lru_cache(maxsize=1)
def _build():
    mesh = plsc.VectorSubcoreMesh(
        core_axis_name="c", subcore_axis_name="s", num_cores=NC, num_subcores=NS
    )

    @functools.partial(
        pl.kernel,
        out_type=jax.ShapeDtypeStruct((NW, NCH, CHUNK), jnp.int32),
        mesh=mesh,
        scratch_types=[
            pltpu.VMEM((NCH, CHUNK), jnp.int32),
            pltpu.VMEM((NCH, CHUNK, 1), jnp.int32),
            pltpu.VMEM((NCH, CHUNK), jnp.int32),
            pltpu.VMEM((NCH, CHUNK, V), jnp.int32),
            pltpu.VMEM((NCH, CHUNK), jnp.int32),
            pltpu.SemaphoreType.DMA,
        ],
    )
    def sc_kernel(tok_hbm, tti_hbm, wcl_hbm, out_hbm,
                  tok_v, ti_v, sidx_v, rows_v, out_v, sem):
        wid = lax.axis_index("s") * NC + lax.axis_index("c")
        pltpu.sync_copy(tok_hbm.at[wid], tok_v)

        cps = [pltpu.async_copy(tti_hbm.at[tok_v.at[j]], ti_v.at[j], sem)
               for j in range(NCH)]
        for cp in cps:
            cp.wait()

        iota = lax.iota(jnp.int32, L)
        zero = jnp.zeros((L,), jnp.int32)
        for g in range(GRP):
            c = g // (CHUNK // L)
            r0 = (g % (CHUNK // L)) * L
            cvec = jnp.full((L,), c, jnp.int32)
            idx = plsc.load_gather(ti_v, [cvec, r0 + iota, zero])
            sidx_v[c, pl.ds(r0, L)] = jnp.maximum(idx, 0)

        cps = [pltpu.async_copy(wcl_hbm.at[sidx_v.at[j]], rows_v.at[j], sem)
               for j in range(NCH)]
        for cp in cps:
            cp.wait()

        for g in range(GRP):
            c = g // (CHUNK // L)
            r0 = (g % (CHUNK // L)) * L
            cvec = jnp.full((L,), c, jnp.int32)
            rvec = r0 + iota
            tok = tok_v[c, pl.ds(r0, L)]
            idx = plsc.load_gather(ti_v, [cvec, rvec, zero])
            best = tok
            for j in range(V - 1, -1, -1):
                col = jnp.full((L,), j, jnp.int32)
                v = plsc.load_gather(rows_v, [cvec, rvec, col])
                best = jnp.where(v != 0, v, best)
            out_v[c, pl.ds(r0, L)] = jnp.where(idx >= 0, best, tok)

        pltpu.sync_copy(out_v, out_hbm.at[wid])

    return sc_kernel


def kernel(input_text, token_to_idx, word_casing_lookup):
    tok32 = input_text.astype(jnp.int32).reshape(NW, NCH, CHUNK)
    tti32 = token_to_idx.astype(jnp.int32).reshape(-1, 1)
    wcl32 = word_casing_lookup.astype(jnp.int32)
    out32 = _build()(tok32, tti32, wcl32)
    return out32.reshape(B).astype(input_text.dtype)


# speedup vs baseline: 1.7192x; 1.7192x over previous
import functools

import jax
import jax.numpy as jnp
from jax import lax
from jax.experimental import pallas as pl
from jax.experimental.pallas import tpu as pltpu
from jax.experimental.pallas import tpu_sc as plsc

B = 16384
V = 8
L = 16
NC, NS = 2, 16
NW = NC * NS
BPW = B // NW
CHUNK = 128
NCH = BPW // CHUNK
GRP = BPW // L


@functools.lru_cache(maxsize=1)
def _build():
    mesh = plsc.VectorSubcoreMesh(
        core_axis_name="c", subcore_axis_name="s", num_cores=NC, num_subcores=NS
    )

    @functools.partial(
        pl.kernel,
        out_type=jax.ShapeDtypeStruct((NW, NCH, CHUNK), jnp.int32),
        mesh=mesh,
        scratch_types=[
            pltpu.VMEM((NCH, CHUNK), jnp.int32),
            pltpu.VMEM((NCH, CHUNK), jnp.int32),
            pltpu.VMEM((NCH, CHUNK), jnp.int32),
            pltpu.VMEM((NCH, CHUNK, V), jnp.int32),
            pltpu.VMEM((NCH, CHUNK), jnp.int32),
            pltpu.SemaphoreType.DMA,
        ],
        compiler_params=pltpu.CompilerParams(
            needs_layout_passes=False, use_tc_tiling_on_sc=False
        ),
    )
    def sc_kernel(tok_hbm, tti_hbm, wcl_hbm, out_hbm,
                  tok_v, ti_v, sidx_v, rows_v, out_v, sem):
        wid = lax.axis_index("s") * NC + lax.axis_index("c")
        pltpu.sync_copy(tok_hbm.at[wid], tok_v)

        cps = [pltpu.async_copy(tti_hbm.at[tok_v.at[jnp.int32(j)]], ti_v.at[jnp.int32(j)], sem)
               for j in range(NCH)]
        for cp in cps:
            cp.wait()

        iota = lax.iota(jnp.int32, L)
        for g in range(GRP):
            c = g // (CHUNK // L)
            r0 = (g % (CHUNK // L)) * L
            idx = ti_v[jnp.int32(c), pl.ds(jnp.int32(r0), L)]
            sidx_v[jnp.int32(c), pl.ds(jnp.int32(r0), L)] = jnp.maximum(idx, 0)

        cps = [pltpu.async_copy(wcl_hbm.at[sidx_v.at[jnp.int32(j)]], rows_v.at[jnp.int32(j)], sem)
               for j in range(NCH)]
        for cp in cps:
            cp.wait()

        for g in range(GRP):
            c = g // (CHUNK // L)
            r0 = (g % (CHUNK // L)) * L
            cvec = jnp.full((L,), c, jnp.int32)
            rvec = jnp.int32(r0) + iota
            tok = tok_v[jnp.int32(c), pl.ds(jnp.int32(r0), L)]
            idx = ti_v[jnp.int32(c), pl.ds(jnp.int32(r0), L)]
            best = tok
            for j in range(V - 1, -1, -1):
                col = jnp.full((L,), j, jnp.int32)
                v = plsc.load_gather(rows_v, [cvec, rvec, col])
                best = jnp.where(v != 0, v, best)
            out_v[jnp.int32(c), pl.ds(jnp.int32(r0), L)] = jnp.where(idx >= 0, best, tok)

        pltpu.sync_copy(out_v, out_hbm.at[wid])

    return sc_kernel


def kernel(input_text, token_to_idx, word_casing_lookup):
    tok32 = input_text.astype(jnp.int32).reshape(NW, NCH, CHUNK)
    tti32 = token_to_idx.astype(jnp.int32)
    wcl32 = word_casing_lookup.astype(jnp.int32)
    out32 = _build()(tok32, tti32, wcl32)
    return out32.reshape(B).astype(input_text.dtype)
